# baseline (speedup 1.0000x reference)
import jax
import jax.numpy as jnp
from jax import lax
from jax.experimental import pallas as pl
from jax.experimental.pallas import tpu as pltpu
from jax.experimental.pallas import tpu_sc as plsc

_N = 10000
_D = 256
_B = 1024
_K = 32
_R = 3
_P = 16
_E = 1024
_C = 2

_NC, _NS, _L = 2, 16, 16
_NW = _NC * _NS

_NEIGH = _R * _B * _K
_NB_W = _NEIGH // _NW
_ND_W = _B // _NW
_SEG = _R * _B
_SEG_W = _SEG // _NW
_SEG_CHUNK = 12
_CHUNKS = _SEG_W // _SEG_CHUNK


def _sc_mesh():
    return plsc.VectorSubcoreMesh(
        core_axis_name="c", subcore_axis_name="s",
        num_cores=_NC, num_subcores=_NS)


_SC_PARAMS = pltpu.CompilerParams(
    needs_layout_passes=False, use_tc_tiling_on_sc=False)


def _scores_body(feat_ref, wl_ref, out_ref):
    out_ref[...] = lax.dot_general(wl_ref[...], feat_ref[...],
                                   (((1,), (1,)), ((), ())),
                                   preferred_element_type=jnp.float32)


def _all_scores(features, w1):
    return pl.pallas_call(
        _scores_body,
        out_shape=jax.ShapeDtypeStruct((1, _N), jnp.float32),
    )(features, w1)


def _fused_body(scores_hbm, neigh_hbm, nodes_hbm, feat_hbm,
                agg_out, srow_out,
                tbl1, ntile, nidx, cidx_a, cidx_b, sidx, srows,
                rows_a, rows_b, aggc, sem_s, sem_a, sem_b):
    wid = lax.axis_index("s") * _NC + lax.axis_index("c")
    pltpu.sync_copy(scores_hbm.at[0], tbl1)
    pltpu.sync_copy(nodes_hbm, ntile)
    pltpu.sync_copy(nodes_hbm.at[pl.ds(wid * _ND_W, _ND_W)], sidx)
    cp_self = pltpu.async_copy(feat_hbm.at[sidx], srows, sem_s)
    pltpu.sync_copy(neigh_hbm.at[pl.ds(wid * _NB_W, _NB_W)], nidx)

    nrow = _SEG_CHUNK * _P
    bufs = ((cidx_a, rows_a, sem_a), (cidx_b, rows_b, sem_b))

    def select_chunk(c):
        cidx = bufs[c % 2][0]

        def seg_sel(i, carry):
            s = c * _SEG_CHUNK + i
            b = lax.rem(wid * _SEG_W + s, _B)
            bvec = jnp.full((_L,), b, jnp.int32)
            nidv = plsc.load_gather(ntile, [bvec])
            sv = plsc.load_gather(tbl1, [nidv])
            ids_a = nidx[pl.ds(s * _K, _L)]
            ids_b = nidx[pl.ds(s * _K + _L, _L)]
            da = jnp.abs(plsc.load_gather(tbl1, [ids_a]) - sv)
            db = jnp.abs(plsc.load_gather(tbl1, [ids_b]) - sv)
            ka, va = plsc.sort_key_val(da, ids_a)
            kb, vb = plsc.sort_key_val(db, ids_b)
            krb = lax.rev(kb, (0,))
            vrb = lax.rev(vb, (0,))
            lo = jnp.where(ka <= krb, va, vrb)
            cidx[pl.ds(i * _P, _P)] = lo
            return carry
        lax.fori_loop(0, _SEG_CHUNK, seg_sel, 0)

    def start(c):
        cidx, rows, sem = bufs[c % 2]
        return pltpu.async_copy(feat_hbm.at[cidx], rows, sem)

    select_chunk(0)
    cps = [start(0)]
    for c in range(_CHUNKS):
        if c + 1 < _CHUNKS:
            select_chunk(c + 1)
            cps.append(start(c + 1))
        cps.pop(0).wait()
        rows = bufs[c % 2][1]

        def seg(s2, carry2):
            def jstep(j, carry3):
                acc = rows[s2 * _P, pl.ds(j * _L, _L)]
                for p in range(1, _P):
                    acc = acc + rows[s2 * _P + p, pl.ds(j * _L, _L)]
                aggc[s2, pl.ds(j * _L, _L)] = acc * (1.0 / _P)
                return carry3
            return lax.fori_loop(0, _D // _L, jstep, carry2)
        lax.fori_loop(0, _SEG_CHUNK, seg, 0)
        pltpu.sync_copy(aggc, agg_out.at[pl.ds(wid * _SEG_W + c * _SEG_CHUNK,
                                               _SEG_CHUNK)])

    cp_self.wait()
    pltpu.sync_copy(srows, srow_out.at[pl.ds(wid * _ND_W, _ND_W)])


def _fused_call(all_scores, neigh_flat, nodes, features):
    fn = pl.kernel(
        _fused_body,
        out_type=(
            jax.ShapeDtypeStruct((_SEG, _D), jnp.float32),
            jax.ShapeDtypeStruct((_B, _D), jnp.float32),
        ),
        mesh=_sc_mesh(),
        scratch_types=[
            pltpu.VMEM((_N,), jnp.float32),
            pltpu.VMEM((_B,), jnp.int32),
            pltpu.VMEM((_NB_W,), jnp.int32),
            pltpu.VMEM((_SEG_CHUNK * _P,), jnp.int32),
            pltpu.VMEM((_SEG_CHUNK * _P,), jnp.int32),
            pltpu.VMEM((_ND_W,), jnp.int32),
            pltpu.VMEM((_ND_W, _D), jnp.float32),
            pltpu.VMEM((_SEG_CHUNK * _P, _D), jnp.float32),
            pltpu.VMEM((_SEG_CHUNK * _P, _D), jnp.float32),
            pltpu.VMEM((_SEG_CHUNK, _D), jnp.float32),
            pltpu.SemaphoreType.DMA,
            pltpu.SemaphoreType.DMA,
            pltpu.SemaphoreType.DMA,
        ],
        compiler_params=_SC_PARAMS,
    )
    return fn(all_scores, neigh_flat, nodes, features)


_EB = _E // 4
_STEPS = 1 + _R * 4 + 1


def _final_body(self_ref, agg_ref, wl_ref, wi_ref, wn_ref, wt_ref,
                out_ref, ls_ref, h_ref):
    t = pl.program_id(0)
    sf = self_ref[...]

    @pl.when(t == 0)
    def _():
        h_ref[...] = jnp.dot(sf, wn_ref[...],
                             preferred_element_type=jnp.float32)

    @pl.when((t >= 1) & (t <= _R * 4))
    def _():
        wr = wi_ref[0]
        ir = jnp.dot(sf, wr[0:_D, :], preferred_element_type=jnp.float32)
        ir = ir + jnp.dot(agg_ref[0], wr[_D:2 * _D, :],
                          preferred_element_type=jnp.float32)
        ir = jnp.maximum(ir, 0.0)
        h_ref[...] = h_ref[...] + jnp.dot(
            ir, wn_ref[...], preferred_element_type=jnp.float32)

    @pl.when(t == _STEPS - 1)
    def _():
        inter = jnp.maximum(h_ref[...], 0.0)
        out_ref[...] = lax.dot_general(inter, wt_ref[...],
                                       (((1,), (1,)), ((), ())),
                                       preferred_element_type=jnp.float32)
        ls_ref[...] = jnp.dot(sf, wl_ref[...],
                              preferred_element_type=jnp.float32)


def _final_call(self_rows, agg3, W_label, W_intra, W_inter, weight):
    def _r(t):
        return jnp.clip((t - 1) // 4, 0, _R - 1)

    def _eb(t):
        return jnp.clip((t - 1) % 4, 0, 3)

    return pl.pallas_call(
        _final_body,
        grid=(_STEPS,),
        in_specs=[
            pl.BlockSpec((_B, _D), lambda t: (0, 0)),
            pl.BlockSpec((1, _B, _D), lambda t: (_r(t), 0, 0)),
            pl.BlockSpec((_D, _C), lambda t: (0, 0)),
            pl.BlockSpec((1, 2 * _D, _EB), lambda t: (_r(t), 0, _eb(t))),
            pl.BlockSpec((_D, _E), lambda t: (jnp.minimum(t, _STEPS - 2), 0)),
            pl.BlockSpec((_C, _E), lambda t: (0, 0)),
        ],
        out_specs=(
            pl.BlockSpec((_B, _C), lambda t: (0, 0)),
            pl.BlockSpec((_B, _C), lambda t: (0, 0)),
        ),
        out_shape=(
            jax.ShapeDtypeStruct((_B, _C), jnp.float32),
            jax.ShapeDtypeStruct((_B, _C), jnp.float32),
        ),
        scratch_shapes=[pltpu.VMEM((_B, _E), jnp.float32)],
    )(self_rows, agg3, W_label, W_intra, W_inter, weight)


def kernel(nodes, labels, neigh_idx, features, train_pos,
           W_label, W_intra, W_inter, weight):
    nodes = nodes.astype(jnp.int32)
    neigh_idx = neigh_idx.astype(jnp.int32)
    score1 = _all_scores(features, W_label[:, 1:2].T)
    agg, self_rows = _fused_call(score1, neigh_idx.reshape(-1),
                                 nodes, features)
    scores, label_scores = _final_call(self_rows, agg.reshape(_R, _B, _D),
                                       W_label, W_intra, W_inter, weight)
    return scores, label_scores


# speedup vs baseline: 1.0370x; 1.0370x over previous
import jax
import jax.numpy as jnp
from jax import lax
from jax.experimental import pallas as pl
from jax.experimental.pallas import tpu as pltpu
from jax.experimental.pallas import tpu_sc as plsc

_N = 10000
_D = 256
_B = 1024
_K = 32
_R = 3
_P = 16
_E = 1024
_C = 2

_NC, _NS, _L = 2, 16, 16
_NW = _NC * _NS

_NEIGH = _R * _B * _K
_NB_W = _NEIGH // _NW
_ND_W = _B // _NW
_SEG = _R * _B
_SEG_W = _SEG // _NW
_SEG_CHUNK = 12
_CHUNKS = _SEG_W // _SEG_CHUNK


def _sc_mesh():
    return plsc.VectorSubcoreMesh(
        core_axis_name="c", subcore_axis_name="s",
        num_cores=_NC, num_subcores=_NS)


_SC_PARAMS = pltpu.CompilerParams(
    needs_layout_passes=False, use_tc_tiling_on_sc=False)


def _scores_body(feat_ref, wl_ref, out_ref):
    out_ref[...] = lax.dot_general(wl_ref[...], feat_ref[...],
                                   (((1,), (1,)), ((), ())),
                                   preferred_element_type=jnp.float32)


def _all_scores(features, w1):
    return pl.pallas_call(
        _scores_body,
        out_shape=jax.ShapeDtypeStruct((1, _N), jnp.float32),
    )(features, w1)


def _fused_body(scores_hbm, neigh_hbm, nodes_hbm, feat_hbm,
                agg_out, srow_out,
                tbl1, ntile, nidx, cidx_a, cidx_b, sidx, srows,
                rows_a, rows_b, aggc, sem_s, sem_a, sem_b):
    wid = lax.axis_index("s") * _NC + lax.axis_index("c")
    pltpu.sync_copy(scores_hbm.at[0], tbl1)
    pltpu.sync_copy(nodes_hbm, ntile)
    pltpu.sync_copy(nodes_hbm.at[pl.ds(wid * _ND_W, _ND_W)], sidx)
    cp_self = pltpu.async_copy(feat_hbm.at[sidx], srows, sem_s)
    pltpu.sync_copy(neigh_hbm.at[pl.ds(wid * _NB_W, _NB_W)], nidx)

    nrow = _SEG_CHUNK * _P
    bufs = ((cidx_a, rows_a, sem_a), (cidx_b, rows_b, sem_b))

    def select_chunk(c):
        cidx = bufs[c % 2][0]

        def seg_sel(i, carry):
            s = c * _SEG_CHUNK + i
            b = lax.rem(wid * _SEG_W + s, _B)
            bvec = jnp.full((_L,), b, jnp.int32)
            nidv = plsc.load_gather(ntile, [bvec])
            sv = plsc.load_gather(tbl1, [nidv])
            ids_a = nidx[pl.ds(s * _K, _L)]
            ids_b = nidx[pl.ds(s * _K + _L, _L)]
            da = jnp.abs(plsc.load_gather(tbl1, [ids_a]) - sv)
            db = jnp.abs(plsc.load_gather(tbl1, [ids_b]) - sv)
            ka, va = plsc.sort_key_val(da, ids_a)
            kb, vb = plsc.sort_key_val(db, ids_b)
            krb = lax.rev(kb, (0,))
            vrb = lax.rev(vb, (0,))
            lo = jnp.where(ka <= krb, va, vrb)
            cidx[pl.ds(i * _P, _P)] = lo
            return carry
        lax.fori_loop(0, _SEG_CHUNK, seg_sel, 0)

    def start(c):
        cidx, rows, sem = bufs[c % 2]
        return pltpu.async_copy(feat_hbm.at[cidx], rows, sem)

    select_chunk(0)
    cps = [start(0)]
    for c in range(_CHUNKS):
        if c + 1 < _CHUNKS:
            select_chunk(c + 1)
            cps.append(start(c + 1))
        cps.pop(0).wait()
        rows = bufs[c % 2][1]

        def seg(s2, carry2):
            def jstep(j, carry3):
                acc = rows[s2 * _P, pl.ds(j * _L, _L)]
                for p in range(1, _P):
                    acc = acc + rows[s2 * _P + p, pl.ds(j * _L, _L)]
                aggc[s2, pl.ds(j * _L, _L)] = acc * (1.0 / _P)
                return carry3
            return lax.fori_loop(0, _D // _L, jstep, carry2)
        lax.fori_loop(0, _SEG_CHUNK, seg, 0)
        pltpu.sync_copy(aggc, agg_out.at[pl.ds(wid * _SEG_W + c * _SEG_CHUNK,
                                               _SEG_CHUNK)])

    cp_self.wait()
    pltpu.sync_copy(srows, srow_out.at[pl.ds(wid * _ND_W, _ND_W)])


def _fused_call(all_scores, neigh_flat, nodes, features):
    fn = pl.kernel(
        _fused_body,
        out_type=(
            jax.ShapeDtypeStruct((_SEG, _D), jnp.float32),
            jax.ShapeDtypeStruct((_B, _D), jnp.float32),
        ),
        mesh=_sc_mesh(),
        scratch_types=[
            pltpu.VMEM((_N,), jnp.float32),
            pltpu.VMEM((_B,), jnp.int32),
            pltpu.VMEM((_NB_W,), jnp.int32),
            pltpu.VMEM((_SEG_CHUNK * _P,), jnp.int32),
            pltpu.VMEM((_SEG_CHUNK * _P,), jnp.int32),
            pltpu.VMEM((_ND_W,), jnp.int32),
            pltpu.VMEM((_ND_W, _D), jnp.float32),
            pltpu.VMEM((_SEG_CHUNK * _P, _D), jnp.float32),
            pltpu.VMEM((_SEG_CHUNK * _P, _D), jnp.float32),
            pltpu.VMEM((_SEG_CHUNK, _D), jnp.float32),
            pltpu.SemaphoreType.DMA,
            pltpu.SemaphoreType.DMA,
            pltpu.SemaphoreType.DMA,
        ],
        compiler_params=_SC_PARAMS,
    )
    return fn(all_scores, neigh_flat, nodes, features)


def _final_body(self_ref, agg_ref, wl_ref, wi_ref, wn_ref, wt_ref,
                out_ref, ls_ref):
    sf = self_ref[...]
    ls_ref[...] = jnp.dot(sf, wl_ref[...],
                          preferred_element_type=jnp.float32)
    h = jnp.dot(sf, wn_ref[0:_D, :], preferred_element_type=jnp.float32)
    for r in range(_R):
        wr = wi_ref[r]
        ir = jnp.dot(sf, wr[0:_D, :], preferred_element_type=jnp.float32)
        ir = ir + jnp.dot(agg_ref[r], wr[_D:2 * _D, :],
                          preferred_element_type=jnp.float32)
        ir = jnp.maximum(ir, 0.0)
        h = h + jnp.dot(ir, wn_ref[_D + r * _E:_D + (r + 1) * _E, :],
                        preferred_element_type=jnp.float32)
    inter = jnp.maximum(h, 0.0)
    out_ref[...] = lax.dot_general(inter, wt_ref[...],
                                   (((1,), (1,)), ((), ())),
                                   preferred_element_type=jnp.float32)


def _final_call(self_rows, agg3, W_label, W_intra, W_inter, weight):
    return pl.pallas_call(
        _final_body,
        out_shape=(
            jax.ShapeDtypeStruct((_B, _C), jnp.float32),
            jax.ShapeDtypeStruct((_B, _C), jnp.float32),
        ),
    )(self_rows, agg3, W_label, W_intra, W_inter, weight)


def kernel(nodes, labels, neigh_idx, features, train_pos,
           W_label, W_intra, W_inter, weight):
    nodes = nodes.astype(jnp.int32)
    neigh_idx = neigh_idx.astype(jnp.int32)
    score1 = _all_scores(features, W_label[:, 1:2].T)
    agg, self_rows = _fused_call(score1, neigh_idx.reshape(-1),
                                 nodes, features)
    scores, label_scores = _final_call(self_rows, agg.reshape(_R, _B, _D),
                                       W_label, W_intra, W_inter, weight)
    return scores, label_scores
